# Initial kernel scaffold; baseline (speedup 1.0000x reference)
import jax
import jax.numpy as jnp
from jax.experimental import pallas as pl


def kernel(x, edge_index, edge_weight, weight):
    raise NotImplementedError("write your pallas kernel here")


# speedup vs baseline: 4.6473x; 4.6473x over previous
import functools

import jax
import jax.numpy as jnp
from jax import lax
from jax.experimental import pallas as pl
from jax.experimental.pallas import tpu as pltpu
from jax.experimental.pallas import tpu_sc as plsc

N_NODES = 10000
N_EDGES = 320000
D = 128

NC = 2
NS = 16
NW = NC * NS
EPW = N_EDGES // NW
K = 80
NCHUNK = EPW // K
N_PAD = 10240
RPT = N_PAD // NS
MM_BLOCK = 2000


def _mm_body(x_ref, w_ref, o_ref):
    o_ref[...] = jnp.dot(x_ref[...], w_ref[...], preferred_element_type=jnp.float32)


def _combine_body(a_ref, b_ref, o_ref):
    o_ref[...] = jnp.maximum(a_ref[...] + b_ref[...], 0.0)


def _sc_spmm(h, src3, dst3, ew3):
    mesh = plsc.VectorSubcoreMesh(core_axis_name="c", subcore_axis_name="s")

    @functools.partial(
        pl.kernel,
        out_type=(
            jax.ShapeDtypeStruct((N_PAD, D), jnp.float32),
            jax.ShapeDtypeStruct((N_PAD, D), jnp.float32),
        ),
        mesh=mesh,
        compiler_params=pltpu.CompilerParams(needs_layout_passes=False),
        scratch_types=(
            pltpu.VMEM((NCHUNK, K), jnp.int32),
            pltpu.VMEM((NCHUNK, K), jnp.int32),
            pltpu.VMEM((K,), jnp.float32),
            pltpu.VMEM((K, D), jnp.float32),
            pltpu.VMEM_SHARED((N_PAD, D), jnp.float32),
            pltpu.SemaphoreType.DMA,
        ),
    )
    def spmm(h_hbm, src_hbm, dst_hbm, ew_hbm, out0, out1,
             src_v, dst_v, ew_c, rows_v, acc, sem):
        cid = lax.axis_index("c")
        sid = lax.axis_index("s")
        w = sid * NC + cid

        def zrow(j, carry):
            for t in range(D // 16):
                rows_v[j, pl.ds(t * 16, 16)] = jnp.zeros((16,), jnp.float32)
            return carry
        lax.fori_loop(0, K, zrow, 0)
        for j in range(RPT // K):
            pltpu.sync_copy(rows_v, acc.at[pl.ds(sid * RPT + j * K, K)])

        pltpu.sync_copy(src_hbm.at[w], src_v)
        pltpu.sync_copy(dst_hbm.at[w], dst_v)

        plsc.subcore_barrier()

        def chunk(i, carry):
            pltpu.sync_copy(ew_hbm.at[pl.ds(w * EPW + i * K, K)], ew_c)
            pltpu.async_copy(h_hbm.at[src_v.at[i]], rows_v, sem).wait()

            def edge(j, c2):
                ws = plsc.load_gather(ew_c, [jnp.full((16,), j, jnp.int32)])
                for t in range(D // 16):
                    rows_v[j, pl.ds(t * 16, 16)] = rows_v[j, pl.ds(t * 16, 16)] * ws
                return c2

            lax.fori_loop(0, K, edge, 0)
            pltpu.sync_copy(rows_v, acc.at[dst_v.at[i]], add=True)
            return carry

        lax.fori_loop(0, NCHUNK, chunk, 0)

        plsc.subcore_barrier()

        @pl.when(cid == 0)
        def _():
            pltpu.sync_copy(acc.at[pl.ds(sid * RPT, RPT)],
                            out0.at[pl.ds(sid * RPT, RPT)])

        @pl.when(cid == 1)
        def _():
            pltpu.sync_copy(acc.at[pl.ds(sid * RPT, RPT)],
                            out1.at[pl.ds(sid * RPT, RPT)])

    return spmm(h, src3, dst3, ew3)


def kernel(x, edge_index, edge_weight, weight):
    x = x.astype(jnp.float32)
    wmat = weight.astype(jnp.float32)
    h = pl.pallas_call(
        _mm_body,
        grid=(N_NODES // MM_BLOCK,),
        in_specs=[pl.BlockSpec((MM_BLOCK, D), lambda i: (i, 0)),
                  pl.BlockSpec((D, D), lambda i: (0, 0))],
        out_specs=pl.BlockSpec((MM_BLOCK, D), lambda i: (i, 0)),
        out_shape=jax.ShapeDtypeStruct((N_NODES, D), jnp.float32),
    )(x, wmat)

    dst3 = edge_index[0].astype(jnp.int32).reshape(NW, NCHUNK, K)
    src3 = edge_index[1].astype(jnp.int32).reshape(NW, NCHUNK, K)
    ew = edge_weight.astype(jnp.float32)
    p0, p1 = _sc_spmm(h, src3, dst3, ew)

    out = pl.pallas_call(
        _combine_body,
        grid=(N_NODES // MM_BLOCK,),
        in_specs=[pl.BlockSpec((MM_BLOCK, D), lambda i: (i, 0)),
                  pl.BlockSpec((MM_BLOCK, D), lambda i: (i, 0))],
        out_specs=pl.BlockSpec((MM_BLOCK, D), lambda i: (i, 0)),
        out_shape=jax.ShapeDtypeStruct((N_NODES, D), jnp.float32),
    )(p0, p1)
    return out


# speedup vs baseline: 8.9210x; 1.9196x over previous
import functools

import jax
import jax.numpy as jnp
from jax import lax
from jax.experimental import pallas as pl
from jax.experimental.pallas import tpu as pltpu
from jax.experimental.pallas import tpu_sc as plsc

N_NODES = 10000
N_EDGES = 320000
D = 128

NC = 2
NS = 16
NW = NC * NS
EPW = N_EDGES // NW
K = 80
NCHUNK = EPW // K
NB = 2
U = 4
N_PAD = 10240
RPT = N_PAD // NS
MM_BLOCK = 2000


def _mm_body(x_ref, w_ref, o_ref):
    o_ref[...] = jnp.dot(x_ref[...], w_ref[...], preferred_element_type=jnp.float32)


def _combine_body(a_ref, b_ref, o_ref):
    o_ref[...] = jnp.maximum(a_ref[...] + b_ref[...], 0.0)


def _sc_spmm(h, src3, dst3, ew3):
    mesh = plsc.VectorSubcoreMesh(core_axis_name="c", subcore_axis_name="s")

    @functools.partial(
        pl.kernel,
        out_type=(
            jax.ShapeDtypeStruct((N_PAD, D), jnp.float32),
            jax.ShapeDtypeStruct((N_PAD, D), jnp.float32),
        ),
        mesh=mesh,
        compiler_params=pltpu.CompilerParams(needs_layout_passes=False),
        scratch_types=(
            pltpu.VMEM((EPW,), jnp.int32),
            pltpu.VMEM((NCHUNK, K), jnp.int32),
            pltpu.VMEM((NB * K,), jnp.float32),
            pltpu.VMEM((NB * K, D), jnp.float32),
            pltpu.VMEM_SHARED((N_PAD, D), jnp.float32),
            pltpu.SemaphoreType.DMA,
            pltpu.SemaphoreType.DMA,
            pltpu.SemaphoreType.DMA,
            pltpu.SemaphoreType.DMA,
        ),
    )
    def spmm(h_hbm, src_hbm, dst_hbm, ew_hbm, out0, out1,
             src_v, dst_v, ew_b, rows_b, acc, sem_r0, sem_r1, sem_e0, sem_e1):
        cid = lax.axis_index("c")
        sid = lax.axis_index("s")
        w = sid * NC + cid
        sem_r = (sem_r0, sem_r1)
        sem_e = (sem_e0, sem_e1)

        def zrow(j, carry):
            for t in range(D // 16):
                rows_b[j, pl.ds(t * 16, 16)] = jnp.zeros((16,), jnp.float32)
            return carry
        lax.fori_loop(0, NB * K, zrow, 0)
        for j in range(RPT // (NB * K)):
            pltpu.sync_copy(rows_b, acc.at[pl.ds(sid * RPT + j * NB * K, NB * K)])

        pltpu.sync_copy(src_hbm.at[pl.ds(w * EPW, EPW)], src_v)
        pltpu.sync_copy(dst_hbm.at[w], dst_v)

        def in_copies(i, b):
            return (
                pltpu.make_async_copy(h_hbm.at[src_v.at[pl.ds(i * K, K)]],
                                      rows_b.at[pl.ds(b * K, K)], sem_r[b]),
                pltpu.make_async_copy(ew_hbm.at[pl.ds(w * EPW + i * K, K)],
                                      ew_b.at[pl.ds(b * K, K)], sem_e[b]),
            )

        def start_in(i, b):
            for c in in_copies(i, b):
                c.start()

        start_in(0, 0)
        plsc.subcore_barrier()

        def outer(g, carry):
            for b in range(NB):
                i = g * NB + b
                nxt = (b + 1) % NB

                @pl.when(i + 1 < NCHUNK)
                def _():
                    start_in(i + 1, nxt)

                @pl.when(i < NCHUNK)
                def _():
                    for c in in_copies(i, b):
                        c.wait()

                    def edge(u, c2):
                        for v in range(U):
                            j = b * K + u * U + v
                            ws = plsc.load_gather(
                                ew_b, [jnp.full((16,), j, jnp.int32)])
                            for t in range(D // 16):
                                rows_b[j, pl.ds(t * 16, 16)] = (
                                    rows_b[j, pl.ds(t * 16, 16)] * ws)
                        return c2

                    lax.fori_loop(0, K // U, edge, 0)
                    pltpu.sync_copy(rows_b.at[pl.ds(b * K, K)],
                                    acc.at[dst_v.at[i]], add=True)
            return carry

        lax.fori_loop(0, (NCHUNK + NB - 1) // NB, outer, 0)

        plsc.subcore_barrier()

        @pl.when(cid == 0)
        def _():
            pltpu.sync_copy(acc.at[pl.ds(sid * RPT, RPT)],
                            out0.at[pl.ds(sid * RPT, RPT)])

        @pl.when(cid == 1)
        def _():
            pltpu.sync_copy(acc.at[pl.ds(sid * RPT, RPT)],
                            out1.at[pl.ds(sid * RPT, RPT)])

    return spmm(h, src3, dst3, ew3)


def kernel(x, edge_index, edge_weight, weight):
    x = x.astype(jnp.float32)
    wmat = weight.astype(jnp.float32)
    h = pl.pallas_call(
        _mm_body,
        grid=(N_NODES // MM_BLOCK,),
        in_specs=[pl.BlockSpec((MM_BLOCK, D), lambda i: (i, 0)),
                  pl.BlockSpec((D, D), lambda i: (0, 0))],
        out_specs=pl.BlockSpec((MM_BLOCK, D), lambda i: (i, 0)),
        out_shape=jax.ShapeDtypeStruct((N_NODES, D), jnp.float32),
    )(x, wmat)

    dst3 = edge_index[0].astype(jnp.int32).reshape(NW, NCHUNK, K)
    src1 = edge_index[1].astype(jnp.int32)
    ew = edge_weight.astype(jnp.float32)
    p0, p1 = _sc_spmm(h, src1, dst3, ew)

    out = pl.pallas_call(
        _combine_body,
        grid=(N_NODES // MM_BLOCK,),
        in_specs=[pl.BlockSpec((MM_BLOCK, D), lambda i: (i, 0)),
                  pl.BlockSpec((MM_BLOCK, D), lambda i: (i, 0))],
        out_specs=pl.BlockSpec((MM_BLOCK, D), lambda i: (i, 0)),
        out_shape=jax.ShapeDtypeStruct((N_NODES, D), jnp.float32),
    )(p0, p1)
    return out


# speedup vs baseline: 10.1200x; 1.1344x over previous
import functools

import jax
import jax.numpy as jnp
from jax import lax
from jax.experimental import pallas as pl
from jax.experimental.pallas import tpu as pltpu
from jax.experimental.pallas import tpu_sc as plsc

N_NODES = 10000
N_EDGES = 320000
D = 128

NC = 2
NS = 16
NW = NC * NS
EPW = N_EDGES // NW
K = 80
NCHUNK = EPW // K
NB = 3
U = 4
N_PAD = 10240
RPT = N_PAD // NS
MM_BLOCK = 2000


def _mm_body(x_ref, w_ref, o_ref):
    o_ref[...] = jnp.dot(x_ref[...], w_ref[...], preferred_element_type=jnp.float32)


def _combine_body(a_ref, b_ref, o_ref):
    o_ref[...] = jnp.maximum(a_ref[...] + b_ref[...], 0.0)


def _sc_spmm(h, src3, dst3, ew3):
    mesh = plsc.VectorSubcoreMesh(core_axis_name="c", subcore_axis_name="s")

    @functools.partial(
        pl.kernel,
        out_type=(
            jax.ShapeDtypeStruct((N_PAD, D), jnp.float32),
            jax.ShapeDtypeStruct((N_PAD, D), jnp.float32),
        ),
        mesh=mesh,
        compiler_params=pltpu.CompilerParams(needs_layout_passes=False),
        scratch_types=(
            pltpu.VMEM((NCHUNK, K), jnp.int32),
            pltpu.VMEM((NB * K,), jnp.int32),
            pltpu.VMEM((NB * K,), jnp.float32),
            pltpu.VMEM((NB * K, D), jnp.float32),
            pltpu.VMEM_SHARED((N_PAD, D), jnp.float32),
        ) + (pltpu.SemaphoreType.DMA,) * (4 * NB),
    )
    def spmm(h_hbm, src_hbm, dst_hbm, ew_hbm, out0, out1,
             dst_v, src_b, ew_b, rows_b, acc, *sems):
        cid = lax.axis_index("c")
        sid = lax.axis_index("s")
        w = sid * NC + cid
        sem_src = sems[0:NB]
        sem_ew = sems[NB:2 * NB]
        sem_r = sems[2 * NB:3 * NB]
        sem_s = sems[3 * NB:4 * NB]

        def zrow(j, carry):
            for t in range(D // 16):
                rows_b[j, pl.ds(t * 16, 16)] = jnp.zeros((16,), jnp.float32)
            return carry
        lax.fori_loop(0, NB * K, zrow, 0)
        pltpu.sync_copy(rows_b, acc.at[pl.ds(sid * RPT, NB * K)])
        pltpu.sync_copy(rows_b, acc.at[pl.ds(sid * RPT + NB * K, NB * K)])
        pltpu.sync_copy(rows_b.at[pl.ds(0, RPT - 2 * NB * K)],
                        acc.at[pl.ds(sid * RPT + 2 * NB * K, RPT - 2 * NB * K)])

        pltpu.sync_copy(dst_hbm.at[w], dst_v)

        def src_copy(i, s):
            return pltpu.make_async_copy(
                src_hbm.at[pl.ds(w * EPW + i * K, K)],
                src_b.at[pl.ds(s * K, K)], sem_src[s])

        def ew_copy(i, s):
            return pltpu.make_async_copy(
                ew_hbm.at[pl.ds(w * EPW + i * K, K)],
                ew_b.at[pl.ds(s * K, K)], sem_ew[s])

        def rows_copy(s):
            return pltpu.make_async_copy(
                h_hbm.at[src_b.at[pl.ds(s * K, K)]],
                rows_b.at[pl.ds(s * K, K)], sem_r[s])

        def scat_copy(i, s):
            return pltpu.make_async_copy(
                rows_b.at[pl.ds(s * K, K)],
                acc.at[dst_v.at[i]], sem_s[s])

        src_copy(0, 0).start()
        ew_copy(0, 0).start()
        src_copy(1, 1).start()
        ew_copy(1, 1).start()
        src_copy(0, 0).wait()
        rows_copy(0).start()
        plsc.subcore_barrier()

        def outer(g, carry):
            for b in range(NB):
                i = g * NB + b
                s1 = (b + 1) % NB
                s2 = (b + 2) % NB

                @pl.when(i + 1 < NCHUNK)
                def _():
                    @pl.when(i >= 2)
                    def _():
                        scat_copy(i - 2, s1).wait()
                    src_copy(i + 1, s1).wait()
                    rows_copy(s1).start()

                @pl.when(i + 2 < NCHUNK)
                def _():
                    src_copy(i + 2, s2).start()
                    ew_copy(i + 2, s2).start()

                @pl.when(i < NCHUNK)
                def _():
                    rows_copy(b).wait()
                    ew_copy(i, b).wait()

                    def edge(u, c2):
                        for v in range(U):
                            j = b * K + u * U + v
                            ws = plsc.load_gather(
                                ew_b, [jnp.full((16,), j, jnp.int32)])
                            for t in range(D // 16):
                                rows_b[j, pl.ds(t * 16, 16)] = (
                                    rows_b[j, pl.ds(t * 16, 16)] * ws)
                        return c2

                    lax.fori_loop(0, K // U, edge, 0)
                    scat_copy(i, b).start(add=True)
            return carry

        lax.fori_loop(0, (NCHUNK + NB - 1) // NB, outer, 0)
        scat_copy(NCHUNK - 3, (NCHUNK - 3) % NB).wait()
        scat_copy(NCHUNK - 2, (NCHUNK - 2) % NB).wait()
        scat_copy(NCHUNK - 1, (NCHUNK - 1) % NB).wait()

        plsc.subcore_barrier()

        @pl.when(cid == 0)
        def _():
            pltpu.sync_copy(acc.at[pl.ds(sid * RPT, RPT)],
                            out0.at[pl.ds(sid * RPT, RPT)])

        @pl.when(cid == 1)
        def _():
            pltpu.sync_copy(acc.at[pl.ds(sid * RPT, RPT)],
                            out1.at[pl.ds(sid * RPT, RPT)])

    return spmm(h, src3, dst3, ew3)


def kernel(x, edge_index, edge_weight, weight):
    x = x.astype(jnp.float32)
    wmat = weight.astype(jnp.float32)
    h = pl.pallas_call(
        _mm_body,
        grid=(N_NODES // MM_BLOCK,),
        in_specs=[pl.BlockSpec((MM_BLOCK, D), lambda i: (i, 0)),
                  pl.BlockSpec((D, D), lambda i: (0, 0))],
        out_specs=pl.BlockSpec((MM_BLOCK, D), lambda i: (i, 0)),
        out_shape=jax.ShapeDtypeStruct((N_NODES, D), jnp.float32),
    )(x, wmat)

    dst3 = edge_index[0].astype(jnp.int32).reshape(NW, NCHUNK, K)
    src1 = edge_index[1].astype(jnp.int32)
    ew = edge_weight.astype(jnp.float32)
    p0, p1 = _sc_spmm(h, src1, dst3, ew)

    out = pl.pallas_call(
        _combine_body,
        grid=(N_NODES // MM_BLOCK,),
        in_specs=[pl.BlockSpec((MM_BLOCK, D), lambda i: (i, 0)),
                  pl.BlockSpec((MM_BLOCK, D), lambda i: (i, 0))],
        out_specs=pl.BlockSpec((MM_BLOCK, D), lambda i: (i, 0)),
        out_shape=jax.ShapeDtypeStruct((N_NODES, D), jnp.float32),
    )(p0, p1)
    return out


# speedup vs baseline: 10.4980x; 1.0374x over previous
import functools

import jax
import jax.numpy as jnp
from jax import lax
from jax.experimental import pallas as pl
from jax.experimental.pallas import tpu as pltpu
from jax.experimental.pallas import tpu_sc as plsc

N_NODES = 10000
N_EDGES = 320000
D = 128

NC = 2
NS = 16
NW = NC * NS
EPW = N_EDGES // NW
K = 80
NCHUNK = EPW // K
NR = 4
NI = 8
U = 8
N_PAD = 10240
RPT = N_PAD // NS
MM_BLOCK = 2000


def _mm_body(x_ref, w_ref, o_ref):
    o_ref[...] = jnp.dot(x_ref[...], w_ref[...], preferred_element_type=jnp.float32)


def _combine_body(a_ref, b_ref, o_ref):
    o_ref[...] = jnp.maximum(a_ref[...] + b_ref[...], 0.0)


def _sc_spmm(h, se, dst1):
    mesh = plsc.VectorSubcoreMesh(core_axis_name="c", subcore_axis_name="s")

    @functools.partial(
        pl.kernel,
        out_type=(
            jax.ShapeDtypeStruct((N_PAD, D), jnp.float32),
            jax.ShapeDtypeStruct((N_PAD, D), jnp.float32),
        ),
        mesh=mesh,
        compiler_params=pltpu.CompilerParams(needs_layout_passes=False),
        scratch_types=(
            pltpu.VMEM((NI * 2 * K,), jnp.int32),
            pltpu.VMEM((NI, K), jnp.int32),
            pltpu.VMEM((NR * K, D), jnp.float32),
            pltpu.VMEM_SHARED((N_PAD, D), jnp.float32),
        ) + (pltpu.SemaphoreType.DMA,) * (NI + 2 * NR),
    )
    def spmm(h_hbm, se_hbm, dst_hbm, out0, out1,
             se_b, dst_b, rows_b, acc, *sems):
        cid = lax.axis_index("c")
        sid = lax.axis_index("s")
        w = sid * NC + cid
        sem_i = sems[0:NI]
        sem_r = sems[NI:NI + NR]
        sem_s = sems[NI + NR:NI + 2 * NR]

        def zrow(j, carry):
            for t in range(D // 16):
                rows_b[j, pl.ds(t * 16, 16)] = jnp.zeros((16,), jnp.float32)
            return carry
        lax.fori_loop(0, NR * K, zrow, 0)
        for j in range(RPT // (NR * K)):
            pltpu.sync_copy(rows_b, acc.at[pl.ds(sid * RPT + j * NR * K, NR * K)])

        def se_copy(i, s):
            return pltpu.make_async_copy(
                se_hbm.at[pl.ds((w * NCHUNK + i) * 2 * K, 2 * K)],
                se_b.at[pl.ds(s * 2 * K, 2 * K)], sem_i[s])

        def dst_copy(i, s):
            return pltpu.make_async_copy(
                dst_hbm.at[pl.ds(w * EPW + i * K, K)],
                dst_b.at[s], sem_i[s])

        def rows_copy(s, si):
            return pltpu.make_async_copy(
                h_hbm.at[se_b.at[pl.ds(si * 2 * K, K)]],
                rows_b.at[pl.ds(s * K, K)], sem_r[s])

        def scat_copy(s, si):
            return pltpu.make_async_copy(
                rows_b.at[pl.ds(s * K, K)],
                acc.at[dst_b.at[si]], sem_s[s])

        def start_idx(i, s):
            se_copy(i, s).start()
            dst_copy(i, s).start()

        def wait_idx(i, s):
            se_copy(i, s).wait()
            dst_copy(i, s).wait()

        start_idx(0, 0)
        start_idx(1, 1)
        start_idx(2, 2)
        wait_idx(0, 0)
        rows_copy(0, 0).start()
        wait_idx(1, 1)
        rows_copy(1, 1).start()
        plsc.subcore_barrier()

        def outer(g, carry):
            for b in range(NI):
                i = g * NI + b
                rb = b % NR
                s2 = (b + 2) % NI
                r2 = (b + 2) % NR
                s3 = (b + 3) % NI

                @pl.when(i + 2 < NCHUNK)
                def _():
                    @pl.when(i >= 2)
                    def _():
                        scat_copy(r2, (b + 6) % NI).wait()
                    wait_idx(i + 2, s2)
                    rows_copy(r2, s2).start()

                @pl.when(i + 3 < NCHUNK)
                def _():
                    start_idx(i + 3, s3)

                @pl.when(i < NCHUNK)
                def _():
                    rows_copy(rb, b).wait()

                    def edge(u, c2):
                        for v in range(U):
                            j = rb * K + u * U + v
                            wi = plsc.load_gather(
                                se_b,
                                [jnp.full((16,), b * 2 * K + K + u * U + v,
                                          jnp.int32)])
                            ws = plsc.bitcast(wi, jnp.float32)
                            for t in range(D // 16):
                                rows_b[j, pl.ds(t * 16, 16)] = (
                                    rows_b[j, pl.ds(t * 16, 16)] * ws)
                        return c2

                    lax.fori_loop(0, K // U, edge, 0)
                    scat_copy(rb, b).start(add=True)
            return carry

        lax.fori_loop(0, (NCHUNK + NI - 1) // NI, outer, 0)
        for x in range(NCHUNK - 4, NCHUNK):
            scat_copy(x % NR, x % NI).wait()

        plsc.subcore_barrier()

        @pl.when(cid == 0)
        def _():
            pltpu.sync_copy(acc.at[pl.ds(sid * RPT, RPT)],
                            out0.at[pl.ds(sid * RPT, RPT)])

        @pl.when(cid == 1)
        def _():
            pltpu.sync_copy(acc.at[pl.ds(sid * RPT, RPT)],
                            out1.at[pl.ds(sid * RPT, RPT)])

    return spmm(h, se, dst1)


def kernel(x, edge_index, edge_weight, weight):
    x = x.astype(jnp.float32)
    wmat = weight.astype(jnp.float32)
    h = pl.pallas_call(
        _mm_body,
        grid=(N_NODES // MM_BLOCK,),
        in_specs=[pl.BlockSpec((MM_BLOCK, D), lambda i: (i, 0)),
                  pl.BlockSpec((D, D), lambda i: (0, 0))],
        out_specs=pl.BlockSpec((MM_BLOCK, D), lambda i: (i, 0)),
        out_shape=jax.ShapeDtypeStruct((N_NODES, D), jnp.float32),
    )(x, wmat)

    dst1 = edge_index[0].astype(jnp.int32)
    src_r = edge_index[1].astype(jnp.int32).reshape(NW, NCHUNK, 1, K)
    ew_r = jax.lax.bitcast_convert_type(
        edge_weight.astype(jnp.float32), jnp.int32).reshape(NW, NCHUNK, 1, K)
    se = jnp.concatenate([src_r, ew_r], axis=2).reshape(-1)
    p0, p1 = _sc_spmm(h, se, dst1)

    out = pl.pallas_call(
        _combine_body,
        grid=(N_NODES // MM_BLOCK,),
        in_specs=[pl.BlockSpec((MM_BLOCK, D), lambda i: (i, 0)),
                  pl.BlockSpec((MM_BLOCK, D), lambda i: (i, 0))],
        out_specs=pl.BlockSpec((MM_BLOCK, D), lambda i: (i, 0)),
        out_shape=jax.ShapeDtypeStruct((N_NODES, D), jnp.float32),
    )(p0, p1)
    return out
